# baseline (speedup 1.0000x reference)
import functools

import jax
import jax.numpy as jnp
from jax import lax
from jax.experimental import pallas as pl


def _body(nxt2_ref, nxtr_ref, xyz_ref, feat_ref, w3_ref, wf_ref, out_ref):
    n = nxtr_ref.shape[2]
    q = n // 4
    i32 = jnp.int32

    sub = nxt2_ref.shape[2]
    lane = nxt2_ref.shape[3]
    c2 = nxt2_ref[0, :, sub - 1 : sub, lane - 1 : lane]
    d2 = nxt2_ref[0] - c2
    d2 = d2 * d2
    dist2 = (d2[0] + d2[1]) + d2[2]
    bits2 = lax.bitcast_convert_type(dist2, i32)

    idx2 = (
        lax.broadcasted_iota(i32, (sub, lane), 0) * lane
        + lax.broadcasted_iota(i32, (sub, lane), 1)
    )

    def cnt(mask):
        return jnp.sum(mask.astype(i32))

    k1, k2, k3 = i32(q), i32(2 * q), i32(3 * q)

    def vstep(i, ts):
        t1, t2, t3 = ts
        bitv = jnp.left_shift(i32(1), i32(30) - i)
        a1, a2, a3 = t1 | bitv, t2 | bitv, t3 | bitv
        t1 = jnp.where(cnt(bits2 < a1) <= k1, a1, t1)
        t2 = jnp.where(cnt(bits2 < a2) <= k2, a2, t2)
        t3 = jnp.where(cnt(bits2 < a3) <= k3, a3, t3)
        return (t1, t2, t3)

    t1, t2, t3 = lax.fori_loop(0, 31, vstep, (i32(0), i32(0), i32(0)))

    eq1, eq2, eq3 = bits2 == t1, bits2 == t2, bits2 == t3
    m1 = k1 - cnt(bits2 < t1)
    m2 = k2 - cnt(bits2 < t2)
    m3 = k3 - cnt(bits2 < t3)

    def istep(i, js):
        j1, j2, j3 = js
        bitv = jnp.left_shift(i32(1), i32(13) - i)
        a1, a2, a3 = j1 | bitv, j2 | bitv, j3 | bitv
        n1 = cnt(eq1 & (idx2 < a1))
        n2 = cnt(eq2 & (idx2 < a2))
        n3 = cnt(eq3 & (idx2 < a3))
        j1 = jnp.where(n1 <= m1, a1, j1)
        j2 = jnp.where(n2 <= m2, a2, j2)
        j3 = jnp.where(n3 <= m3, a3, j3)
        return (j1, j2, j3)

    i1, i2, i3 = lax.fori_loop(0, 14, istep, (i32(0), i32(0), i32(0)))

    cr = nxtr_ref[0, :, n - 1 : n]
    dr = nxtr_ref[0] - cr
    dr = dr * dr
    distr = (dr[0:1] + dr[1:2]) + dr[2:3]
    bitsr = lax.bitcast_convert_type(distr, i32)
    idxr = lax.broadcasted_iota(i32, (1, n), 1)

    def ge(tg, ig):
        return (bitsr > tg) | ((bitsr == tg) & (idxr >= ig))

    grp = ge(t1, i1).astype(i32) + ge(t2, i2).astype(i32) + ge(t3, i3).astype(i32)

    nf = lax.dot_general(
        wf_ref[...], feat_ref[0],
        (((1,), (0,)), ((), ())),
        preferred_element_type=jnp.float32,
    ) + lax.dot_general(
        w3_ref[...], jnp.transpose(xyz_ref[0], (1, 0)),
        (((1,), (0,)), ((), ())),
        preferred_element_type=jnp.float32,
    )
    neg = jnp.float32(-jnp.inf)
    cols = []
    for g in range(4):
        cols.append(jnp.max(jnp.where(grp == i32(g), nf, neg), axis=1, keepdims=True))
    out_ref[0] = jnp.concatenate(cols, axis=1)


@jax.jit
def kernel(xyz, new_xyz, features, W):
    b, n, _ = xyz.shape
    cfeat = features.shape[1]
    cout = W.shape[0]

    nxt = jnp.transpose(new_xyz, (0, 2, 1))
    nxt2 = nxt.reshape(b, 3, 8, n // 8)
    w3 = W[:, :3]
    wf = W[:, 3:]

    out = pl.pallas_call(
        _body,
        grid=(b,),
        in_specs=[
            pl.BlockSpec((1, 3, 8, n // 8), lambda i: (i, 0, 0, 0)),
            pl.BlockSpec((1, 3, n), lambda i: (i, 0, 0)),
            pl.BlockSpec((1, n, 3), lambda i: (i, 0, 0)),
            pl.BlockSpec((1, cfeat, n), lambda i: (i, 0, 0)),
            pl.BlockSpec((cout, 3), lambda i: (0, 0)),
            pl.BlockSpec((cout, cfeat), lambda i: (0, 0)),
        ],
        out_specs=pl.BlockSpec((1, cout, 4), lambda i: (i, 0, 0)),
        out_shape=jax.ShapeDtypeStruct((b, cout, 4), jnp.float32),
    )(nxt2, nxt, xyz, features, w3, wf)
    return out


# speedup vs baseline: 1.2713x; 1.2713x over previous
import functools

import jax
import jax.numpy as jnp
from jax import lax
from jax.experimental import pallas as pl


def _body(nxt2_ref, nxtr_ref, xyzt_ref, feat_ref, w3_ref, wf_ref, out_ref):
    n = nxtr_ref.shape[2]
    q = n // 4
    i32 = jnp.int32

    sub = nxt2_ref.shape[2]
    lane = nxt2_ref.shape[3]
    c2 = nxt2_ref[0, :, sub - 1 : sub, lane - 1 : lane]
    d2 = nxt2_ref[0] - c2
    d2 = d2 * d2
    dist2 = (d2[0] + d2[1]) + d2[2]
    bits2 = lax.bitcast_convert_type(dist2, i32)

    idx2 = (
        lax.broadcasted_iota(i32, (sub, lane), 0) * lane
        + lax.broadcasted_iota(i32, (sub, lane), 1)
    )

    def cnt(mask):
        return jnp.sum(mask.astype(i32))

    k1, k2, k3 = i32(q), i32(2 * q), i32(3 * q)

    def vstep(i, ts):
        t1, t2, t3 = ts
        bitv = jnp.left_shift(i32(1), i32(30) - i)
        a1, a2, a3 = t1 | bitv, t2 | bitv, t3 | bitv
        t1 = jnp.where(cnt(bits2 < a1) <= k1, a1, t1)
        t2 = jnp.where(cnt(bits2 < a2) <= k2, a2, t2)
        t3 = jnp.where(cnt(bits2 < a3) <= k3, a3, t3)
        return (t1, t2, t3)

    t1, t2, t3 = lax.fori_loop(0, 31, vstep, (i32(0), i32(0), i32(0)))

    eq1, eq2, eq3 = bits2 == t1, bits2 == t2, bits2 == t3
    m1 = k1 - cnt(bits2 < t1)
    m2 = k2 - cnt(bits2 < t2)
    m3 = k3 - cnt(bits2 < t3)

    def istep(i, js):
        j1, j2, j3 = js
        bitv = jnp.left_shift(i32(1), i32(13) - i)
        a1, a2, a3 = j1 | bitv, j2 | bitv, j3 | bitv
        n1 = cnt(eq1 & (idx2 < a1))
        n2 = cnt(eq2 & (idx2 < a2))
        n3 = cnt(eq3 & (idx2 < a3))
        j1 = jnp.where(n1 <= m1, a1, j1)
        j2 = jnp.where(n2 <= m2, a2, j2)
        j3 = jnp.where(n3 <= m3, a3, j3)
        return (j1, j2, j3)

    i1, i2, i3 = lax.fori_loop(0, 14, istep, (i32(0), i32(0), i32(0)))

    cr = nxtr_ref[0, :, n - 1 : n]
    dr = nxtr_ref[0] - cr
    dr = dr * dr
    distr = (dr[0:1] + dr[1:2]) + dr[2:3]
    bitsr = lax.bitcast_convert_type(distr, i32)
    idxr = lax.broadcasted_iota(i32, (1, n), 1)

    def ge(tg, ig):
        return (bitsr > tg) | ((bitsr == tg) & (idxr >= ig))

    grp = ge(t1, i1).astype(i32) + ge(t2, i2).astype(i32) + ge(t3, i3).astype(i32)

    nf = lax.dot_general(
        wf_ref[...], feat_ref[0],
        (((1,), (0,)), ((), ())),
        preferred_element_type=jnp.float32,
    ) + lax.dot_general(
        w3_ref[...], xyzt_ref[0],
        (((1,), (0,)), ((), ())),
        preferred_element_type=jnp.float32,
    )
    neg = jnp.float32(-jnp.inf)
    cols = []
    for g in range(4):
        cols.append(jnp.max(jnp.where(grp == i32(g), nf, neg), axis=1, keepdims=True))
    out_ref[0] = jnp.concatenate(cols, axis=1)


@jax.jit
def kernel(xyz, new_xyz, features, W):
    b, n, _ = xyz.shape
    cfeat = features.shape[1]
    cout = W.shape[0]

    nxt = jnp.transpose(new_xyz, (0, 2, 1))
    xyzt = jnp.transpose(xyz, (0, 2, 1))
    nxt2 = nxt.reshape(b, 3, 8, n // 8)
    w3 = W[:, :3]
    wf = W[:, 3:]

    out = pl.pallas_call(
        _body,
        grid=(b,),
        in_specs=[
            pl.BlockSpec((1, 3, 8, n // 8), lambda i: (i, 0, 0, 0)),
            pl.BlockSpec((1, 3, n), lambda i: (i, 0, 0)),
            pl.BlockSpec((1, 3, n), lambda i: (i, 0, 0)),
            pl.BlockSpec((1, cfeat, n), lambda i: (i, 0, 0)),
            pl.BlockSpec((cout, 3), lambda i: (0, 0)),
            pl.BlockSpec((cout, cfeat), lambda i: (0, 0)),
        ],
        out_specs=pl.BlockSpec((1, cout, 4), lambda i: (i, 0, 0)),
        out_shape=jax.ShapeDtypeStruct((b, cout, 4), jnp.float32),
    )(nxt2, nxt, xyzt, features, w3, wf)
    return out
